# baseline (speedup 1.0000x reference)
import functools

import jax
import jax.numpy as jnp
from jax import lax
from jax.experimental import pallas as pl
from jax.experimental.pallas import tpu as pltpu
from jax.experimental.pallas import tpu_sc as plsc

NC = 2
NS = 16
L = 16
NW = NC * NS

def _take(x, perm):
    return x.at[perm].get(mode="promise_in_bounds")


def _make_perms():
    l = lax.iota(jnp.int32, L)
    p1a = l & 7
    p2a = (l & 3) | ((l & 4) << 1)
    p3a = (l & 1) | ((l & 6) << 1)
    p4a = (l & 7) << 1
    return ((p1a, p1a + 8), (p2a, p2a + 4), (p3a, p3a + 2), (p4a, p4a + 1))


def _merge(a, b, pa, pb, mask_lo):
    return jnp.where(mask_lo, _take(a, pa) + _take(a, pb),
                     _take(b, pa) + _take(b, pb))


def _make_sc_kernel(B, F, V, E, C):
    assert E == L
    RPW = B // NW
    CHUNKS = RPW // C
    IPC = C * F // 128
    assert C * F % 128 == 0 and RPW % C == 0
    IROWS_PW = RPW * F // 128

    mesh = plsc.VectorSubcoreMesh(core_axis_name="c", subcore_axis_name="s")

    @functools.partial(
        pl.kernel,
        mesh=mesh,
        compiler_params=pltpu.CompilerParams(use_tc_tiling_on_sc=False),
        out_type=jax.ShapeDtypeStruct((B,), jnp.float32),
        scratch_types=[
            pltpu.VMEM((IROWS_PW, 128), jnp.int32),
            pltpu.VMEM((C * F, E), jnp.float32),
            pltpu.VMEM((C, L), jnp.float32),
            pltpu.VMEM((C,), jnp.float32),
            pltpu.VMEM((L,), jnp.float32),
            pltpu.SemaphoreType.DMA,
        ],
    )
    def sc_kernel(tab_h, idx_h, dense_h, fcw_h, out_h,
                  idx_v, buf, dense_v, out_v, fcw_v, sem):
        wid = lax.axis_index("s") * NC + lax.axis_index("c")
        pltpu.sync_copy(fcw_h, fcw_v)
        pltpu.sync_copy(idx_h.at[pl.ds(wid * IROWS_PW, IROWS_PW)], idx_v)
        fcw = fcw_v[...]
        perms = _make_perms()
        mask_lo = lax.iota(jnp.int32, L) < 8

        def chunk_body(k, _):
            r0 = wid * RPW + k * C
            pltpu.sync_copy(dense_h.at[pl.ds(r0, C)], dense_v)
            for j in range(IPC):
                pltpu.make_async_copy(
                    tab_h.at[idx_v.at[k * IPC + j]],
                    buf.at[pl.ds(j * 128, 128)], sem,
                ).start()
            for j in range(IPC):
                pltpu.make_async_copy(
                    tab_h.at[idx_v.at[k * IPC + j]],
                    buf.at[pl.ds(j * 128, 128)], sem,
                ).wait()

            def group_body(g, _):
                vs = []
                for t in range(L):
                    i = g * L + t
                    acc = dense_v[i] * fcw
                    for f in range(F):
                        acc = acc + buf[i * F + f]
                    vs.append(acc)
                for pa, pb in perms:
                    vs = [_merge(vs[2 * p], vs[2 * p + 1], pa, pb, mask_lo)
                          for p in range(len(vs) // 2)]
                out_v[pl.ds(g * L, L)] = vs[0]
                return _

            lax.fori_loop(0, C // L, group_body, None)
            pltpu.sync_copy(out_v, out_h.at[pl.ds(r0, C)])
            return _

        lax.fori_loop(0, CHUNKS, chunk_body, None)

    return sc_kernel


def kernel(sparse_idx, dense, tables, fc_w, fc_b):
    B, F = sparse_idx.shape
    Ft, V, E = tables.shape
    D = dense.shape[1]

    tab2d = tables.reshape(F * V, E)
    idx32 = sparse_idx.astype(jnp.int32) + (
        jnp.arange(F, dtype=jnp.int32) * V
    )[None, :]
    idx2d = idx32.reshape(B * F // 128, 128)

    pad = jnp.zeros((B, L - D - 1), jnp.float32)
    dense_ext = jnp.concatenate(
        [dense, jnp.ones((B, 1), jnp.float32), pad], axis=1)
    fcw_ext = jnp.concatenate(
        [fc_w[:, 0], fc_b, jnp.zeros((L - D - 1,), jnp.float32)])

    sc = _make_sc_kernel(B, F, V, E, C=64)
    out = sc(tab2d, idx2d, dense_ext, fcw_ext)
    return out[:, None]


# speedup vs baseline: 3.7857x; 3.7857x over previous
import functools

import jax
import jax.numpy as jnp
from jax import lax
from jax.experimental import pallas as pl
from jax.experimental.pallas import tpu as pltpu
from jax.experimental.pallas import tpu_sc as plsc

NC = 2
NS = 16
L = 16
NW = NC * NS

CHUNK = 8192
VMAIN = 99968
NVC = 13
TAIL0 = 12 * CHUNK
TAILN = VMAIN - TAIL0


def _make_phase1(F, V, E):
    NU = F * NVC
    KMAX = (NU + NW - 1) // NW
    mesh = plsc.VectorSubcoreMesh(core_axis_name="c", subcore_axis_name="s")

    @functools.partial(
        pl.kernel,
        mesh=mesh,
        compiler_params=pltpu.CompilerParams(
            use_tc_tiling_on_sc=True, needs_layout_passes=False),
        out_type=jax.ShapeDtypeStruct((F * VMAIN,), jnp.float32),
        scratch_types=[
            pltpu.VMEM((8, CHUNK), jnp.float32),
            pltpu.VMEM((CHUNK,), jnp.float32),
        ],
    )
    def phase1(tab_h, s_h, slab_v, acc_v):
        wid = lax.axis_index("s") * NC + lax.axis_index("c")

        def do_chunk(f, v0, n):
            for dg in range(2):
                pltpu.sync_copy(
                    tab_h.at[f, pl.ds(dg * 8, 8), pl.ds(v0, n)],
                    slab_v.at[:, pl.ds(0, n)])

                def red(j, _, dg=dg):
                    sl = pl.ds(j * L, L)
                    a = slab_v[0, sl]
                    for d in range(1, 8):
                        a = a + slab_v[d, sl]
                    if dg == 0:
                        acc_v[sl] = a
                    else:
                        acc_v[sl] = acc_v[sl] + a
                    return _

                lax.fori_loop(0, n // L, red, None)
            pltpu.sync_copy(acc_v.at[pl.ds(0, n)],
                            s_h.at[pl.ds(f * VMAIN + v0, n)])

        for k in range(KMAX):
            uid = wid + NW * k
            f = uid // NVC
            vc = uid - f * NVC

            @pl.when(jnp.logical_and(uid < NU, vc < NVC - 1))
            def _():
                do_chunk(f, vc * CHUNK, CHUNK)

            @pl.when(jnp.logical_and(uid < NU, vc == NVC - 1))
            def _():
                do_chunk(f, TAIL0, TAILN)

    return phase1


def _make_phase2(F, V, B, D):
    RPW = B // NW
    NVEC = RPW // L
    NST = F * RPW // 128
    NTAIL = V - VMAIN
    mesh = plsc.VectorSubcoreMesh(core_axis_name="c", subcore_axis_name="s")

    @functools.partial(
        pl.kernel,
        mesh=mesh,
        compiler_params=pltpu.CompilerParams(
            use_tc_tiling_on_sc=False, needs_layout_passes=False),
        out_type=jax.ShapeDtypeStruct((B,), jnp.float32),
        scratch_types=[
            pltpu.VMEM((F * RPW,), jnp.int32),
            pltpu.VMEM((F * RPW,), jnp.int32),
            pltpu.VMEM((F * RPW,), jnp.float32),
            pltpu.VMEM((D * RPW,), jnp.float32),
            pltpu.VMEM(((D + 1) * L,), jnp.float32),
            pltpu.VMEM((F * NTAIL,), jnp.float32),
            pltpu.VMEM((RPW,), jnp.float32),
            pltpu.SemaphoreType.DMA,
        ],
    )
    def phase2(s_h, stail_h, idx_h, den_h, w_h, out_h,
               idx_v, pos_v, buf_v, den_v, w_v, stail_v, out_v, sem):
        wid = lax.axis_index("s") * NC + lax.axis_index("c")
        b0 = wid * RPW
        pltpu.sync_copy(w_h, w_v)
        pltpu.sync_copy(stail_h, stail_v)
        for f in range(F):
            pltpu.sync_copy(idx_h.at[pl.ds(f * B + b0, RPW)],
                            idx_v.at[pl.ds(f * RPW, RPW)])
        for dd in range(D):
            pltpu.sync_copy(den_h.at[pl.ds(dd * B + b0, RPW)],
                            den_v.at[pl.ds(dd * RPW, RPW)])

        vmax = jnp.full((L,), VMAIN - 1, jnp.int32)
        for f in range(F):
            off = jnp.full((L,), f * VMAIN, jnp.int32)

            def mkpos(j, _, off=off, f=f):
                sl = pl.ds(f * RPW + j * L, L)
                pos_v[sl] = jnp.minimum(idx_v[sl], vmax) + off
                return _

            lax.fori_loop(0, NVEC, mkpos, None)

        def fire(bi, _):
            for j in range(13):
                st = (bi * 13 + j) * 128
                pltpu.make_async_copy(
                    s_h.at[pos_v.at[pl.ds(st, 128)]],
                    buf_v.at[pl.ds(st, 128)], sem).start()
            for j in range(13):
                st = (bi * 13 + j) * 128
                pltpu.make_async_copy(
                    s_h.at[pos_v.at[pl.ds(st, 128)]],
                    buf_v.at[pl.ds(st, 128)], sem).wait()
            return _

        lax.fori_loop(0, NST // 13, fire, None)

        wvecs = [w_v[pl.ds(dd * L, L)] for dd in range(D + 1)]
        vlim = jnp.full((L,), VMAIN, jnp.int32)
        zero = jnp.zeros((L,), jnp.int32)

        def red(g, _):
            acc = wvecs[D]
            for dd in range(D):
                acc = acc + den_v[pl.ds(dd * RPW + g * L, L)] * wvecs[dd]
            for f in range(F):
                sl = pl.ds(f * RPW + g * L, L)
                v = idx_v[sl]
                tv = plsc.load_gather(
                    stail_v, [jnp.maximum(v - vlim, zero) + (f * NTAIL)])
                acc = acc + jnp.where(v >= vlim, tv, buf_v[sl])
            out_v[pl.ds(g * L, L)] = acc
            return _

        lax.fori_loop(0, NVEC, red, None)
        pltpu.sync_copy(out_v, out_h.at[pl.ds(b0, RPW)])

    return phase2


def kernel(sparse_idx, dense, tables, fc_w, fc_b):
    B, F = sparse_idx.shape
    Ft, V, E = tables.shape
    D = dense.shape[1]

    tab_t = jnp.swapaxes(tables, 1, 2)
    idx_flat = sparse_idx.astype(jnp.int32).T.reshape(-1)
    den_flat = dense.T.reshape(-1)
    w_splat = jnp.concatenate(
        [jnp.repeat(fc_w[:, 0], L), jnp.repeat(fc_b, L)])
    s_tail = jnp.sum(tables[:, VMAIN:, :], axis=-1).reshape(-1)

    s_flat = _make_phase1(F, V, E)(tab_t)
    out = _make_phase2(F, V, B, D)(
        s_flat, s_tail, idx_flat, den_flat, w_splat)
    return out[:, None]


# speedup vs baseline: 4.4078x; 1.1643x over previous
import functools

import jax
import jax.numpy as jnp
from jax import lax
from jax.experimental import pallas as pl
from jax.experimental.pallas import tpu as pltpu
from jax.experimental.pallas import tpu_sc as plsc

NC = 2
NS = 16
L = 16
NW = NC * NS

CHUNK = 8192
SUB = 2048
NSUB = CHUNK // SUB
VMAIN = 99968
NVC = 13
TAIL0 = 12 * CHUNK
TAILN = VMAIN - TAIL0


def _make_phase1(F, V, E):
    NU = F * NVC
    KMAX = (NU + NW - 1) // NW
    mesh = plsc.VectorSubcoreMesh(core_axis_name="c", subcore_axis_name="s")

    @functools.partial(
        pl.kernel,
        mesh=mesh,
        compiler_params=pltpu.CompilerParams(
            use_tc_tiling_on_sc=True, needs_layout_passes=False),
        out_type=jax.ShapeDtypeStruct((F * VMAIN,), jnp.float32),
        scratch_types=[
            pltpu.VMEM((2, 8, SUB), jnp.float32),
            pltpu.VMEM((CHUNK,), jnp.float32),
            pltpu.SemaphoreType.DMA,
            pltpu.SemaphoreType.DMA,
        ],
    )
    def phase1(tab_h, s_h, slab_v, acc_v, sem0, sem1):
        wid = lax.axis_index("s") * NC + lax.axis_index("c")
        sems = (sem0, sem1)

        def do_chunk(f, v0):
            def cp(i):
                dg, sub = divmod(i, NSUB)
                return pltpu.make_async_copy(
                    tab_h.at[f, pl.ds(dg * 8, 8),
                             pl.ds(v0 + sub * SUB, SUB)],
                    slab_v.at[i % 2], sems[i % 2])

            cp(0).start()
            for i in range(2 * NSUB):
                if i + 1 < 2 * NSUB:
                    cp(i + 1).start()
                cp(i).wait()
                dg, sub = divmod(i, NSUB)
                buf = i % 2

                def red(j, _, dg=dg, sub=sub, buf=buf):
                    sl = pl.ds(j * L, L)
                    a = slab_v[buf, 0, sl]
                    for d in range(1, 8):
                        a = a + slab_v[buf, d, sl]
                    asl = pl.ds(sub * SUB + j * L, L)
                    if dg == 0:
                        acc_v[asl] = a
                    else:
                        acc_v[asl] = acc_v[asl] + a
                    return _

                lax.fori_loop(0, SUB // L, red, None)
            pltpu.sync_copy(acc_v, s_h.at[pl.ds(f * VMAIN + v0, CHUNK)])

        def do_tail(f, v0, n):
            for dg in range(2):
                pltpu.make_async_copy(
                    tab_h.at[f, pl.ds(dg * 8, 8), pl.ds(v0, n)],
                    slab_v.at[0, :, pl.ds(0, n)], sem0).start()
                pltpu.make_async_copy(
                    tab_h.at[f, pl.ds(dg * 8, 8), pl.ds(v0, n)],
                    slab_v.at[0, :, pl.ds(0, n)], sem0).wait()

                def red(j, _, dg=dg):
                    sl = pl.ds(j * L, L)
                    a = slab_v[0, 0, sl]
                    for d in range(1, 8):
                        a = a + slab_v[0, d, sl]
                    if dg == 0:
                        acc_v[sl] = a
                    else:
                        acc_v[sl] = acc_v[sl] + a
                    return _

                lax.fori_loop(0, n // L, red, None)
            pltpu.sync_copy(acc_v.at[pl.ds(0, n)],
                            s_h.at[pl.ds(f * VMAIN + v0, n)])

        for k in range(KMAX):
            uid = wid + NW * k
            f = uid // NVC
            vc = uid - f * NVC

            @pl.when(jnp.logical_and(uid < NU, vc < NVC - 1))
            def _():
                do_chunk(f, vc * CHUNK)

            @pl.when(jnp.logical_and(uid < NU, vc == NVC - 1))
            def _():
                do_tail(f, TAIL0, TAILN)

    return phase1


def _make_phase2(F, V, B, D):
    RPW = B // NW
    NVEC = RPW // L
    NST = F * RPW // 128
    NTAIL = V - VMAIN
    mesh = plsc.VectorSubcoreMesh(core_axis_name="c", subcore_axis_name="s")

    @functools.partial(
        pl.kernel,
        mesh=mesh,
        compiler_params=pltpu.CompilerParams(
            use_tc_tiling_on_sc=False, needs_layout_passes=False),
        out_type=jax.ShapeDtypeStruct((B,), jnp.float32),
        scratch_types=[
            pltpu.VMEM((F * RPW,), jnp.int32),
            pltpu.VMEM((F * RPW,), jnp.int32),
            pltpu.VMEM((F * RPW,), jnp.float32),
            pltpu.VMEM((D * RPW,), jnp.float32),
            pltpu.VMEM(((D + 1) * L,), jnp.float32),
            pltpu.VMEM((F * NTAIL,), jnp.float32),
            pltpu.VMEM((RPW,), jnp.float32),
            pltpu.SemaphoreType.DMA,
        ],
    )
    def phase2(s_h, stail_h, idx_h, den_h, w_h, out_h,
               idx_v, pos_v, buf_v, den_v, w_v, stail_v, out_v, sem):
        wid = lax.axis_index("s") * NC + lax.axis_index("c")
        b0 = wid * RPW
        pltpu.sync_copy(w_h, w_v)
        pltpu.sync_copy(stail_h, stail_v)
        for f in range(F):
            pltpu.sync_copy(idx_h.at[pl.ds(f * B + b0, RPW)],
                            idx_v.at[pl.ds(f * RPW, RPW)])
        for dd in range(D):
            pltpu.sync_copy(den_h.at[pl.ds(dd * B + b0, RPW)],
                            den_v.at[pl.ds(dd * RPW, RPW)])

        vmax = jnp.full((L,), VMAIN - 1, jnp.int32)
        for f in range(F):
            off = jnp.full((L,), f * VMAIN, jnp.int32)

            def mkpos(j, _, off=off, f=f):
                sl = pl.ds(f * RPW + j * L, L)
                pos_v[sl] = jnp.minimum(idx_v[sl], vmax) + off
                return _

            lax.fori_loop(0, NVEC, mkpos, None)

        def fire(bi, _):
            for j in range(13):
                st = (bi * 13 + j) * 128
                pltpu.make_async_copy(
                    s_h.at[pos_v.at[pl.ds(st, 128)]],
                    buf_v.at[pl.ds(st, 128)], sem).start()
            for j in range(13):
                st = (bi * 13 + j) * 128
                pltpu.make_async_copy(
                    s_h.at[pos_v.at[pl.ds(st, 128)]],
                    buf_v.at[pl.ds(st, 128)], sem).wait()
            return _

        lax.fori_loop(0, NST // 13, fire, None)

        wvecs = [w_v[pl.ds(dd * L, L)] for dd in range(D + 1)]
        vlim = jnp.full((L,), VMAIN, jnp.int32)
        zero = jnp.zeros((L,), jnp.int32)

        def red(g, _):
            acc = wvecs[D]
            for dd in range(D):
                acc = acc + den_v[pl.ds(dd * RPW + g * L, L)] * wvecs[dd]
            for f in range(F):
                sl = pl.ds(f * RPW + g * L, L)
                v = idx_v[sl]
                tv = plsc.load_gather(
                    stail_v, [jnp.maximum(v - vlim, zero) + (f * NTAIL)])
                acc = acc + jnp.where(v >= vlim, tv, buf_v[sl])
            out_v[pl.ds(g * L, L)] = acc
            return _

        lax.fori_loop(0, NVEC, red, None)
        pltpu.sync_copy(out_v, out_h.at[pl.ds(b0, RPW)])

    return phase2


def kernel(sparse_idx, dense, tables, fc_w, fc_b):
    B, F = sparse_idx.shape
    Ft, V, E = tables.shape
    D = dense.shape[1]

    tab_t = jnp.swapaxes(tables, 1, 2)
    idx_flat = sparse_idx.astype(jnp.int32).T.reshape(-1)
    den_flat = dense.T.reshape(-1)
    w_splat = jnp.concatenate(
        [jnp.repeat(fc_w[:, 0], L), jnp.repeat(fc_b, L)])
    s_tail = jnp.sum(tables[:, VMAIN:, :], axis=-1).reshape(-1)

    s_flat = _make_phase1(F, V, E)(tab_t)
    out = _make_phase2(F, V, B, D)(
        s_flat, s_tail, idx_flat, den_flat, w_splat)
    return out[:, None]


# speedup vs baseline: 4.7767x; 1.0837x over previous
import functools

import jax
import jax.numpy as jnp
from jax import lax
from jax.experimental import pallas as pl
from jax.experimental.pallas import tpu as pltpu
from jax.experimental.pallas import tpu_sc as plsc

NC = 2
NS = 16
L = 16
NW = NC * NS

CHUNK = 8192
SUB = 2048
NSUB = CHUNK // SUB
VMAIN = 99968
NVC = 13
TAIL0 = 12 * CHUNK
TAILN = VMAIN - TAIL0


def _make_phase1(F, V, E):
    NU = F * NVC
    KMAX = (NU + NW - 1) // NW
    mesh = plsc.VectorSubcoreMesh(core_axis_name="c", subcore_axis_name="s")

    @functools.partial(
        pl.kernel,
        mesh=mesh,
        compiler_params=pltpu.CompilerParams(
            use_tc_tiling_on_sc=True, needs_layout_passes=False),
        out_type=jax.ShapeDtypeStruct((F * VMAIN,), jnp.float32),
        scratch_types=[
            pltpu.VMEM((2, 8, SUB), jnp.float32),
            pltpu.VMEM((CHUNK,), jnp.float32),
            pltpu.SemaphoreType.DMA,
            pltpu.SemaphoreType.DMA,
        ],
    )
    def phase1(tab_h, s_h, slab_v, acc_v, sem0, sem1):
        wid = lax.axis_index("s") * NC + lax.axis_index("c")
        sems = (sem0, sem1)

        def do_chunk(f, v0):
            def cp(i):
                dg, sub = divmod(i, NSUB)
                return pltpu.make_async_copy(
                    tab_h.at[f, pl.ds(dg * 8, 8),
                             pl.ds(v0 + sub * SUB, SUB)],
                    slab_v.at[i % 2], sems[i % 2])

            cp(0).start()
            for i in range(2 * NSUB):
                if i + 1 < 2 * NSUB:
                    cp(i + 1).start()
                cp(i).wait()
                dg, sub = divmod(i, NSUB)
                buf = i % 2

                def red(j, _, dg=dg, sub=sub, buf=buf):
                    sl = pl.ds(j * L, L)
                    a = slab_v[buf, 0, sl]
                    for d in range(1, 8):
                        a = a + slab_v[buf, d, sl]
                    asl = pl.ds(sub * SUB + j * L, L)
                    if dg == 0:
                        acc_v[asl] = a
                    else:
                        acc_v[asl] = acc_v[asl] + a
                    return _

                lax.fori_loop(0, SUB // L, red, None)
            pltpu.sync_copy(acc_v, s_h.at[pl.ds(f * VMAIN + v0, CHUNK)])

        def do_tail(f, v0, n):
            for dg in range(2):
                pltpu.make_async_copy(
                    tab_h.at[f, pl.ds(dg * 8, 8), pl.ds(v0, n)],
                    slab_v.at[0, :, pl.ds(0, n)], sem0).start()
                pltpu.make_async_copy(
                    tab_h.at[f, pl.ds(dg * 8, 8), pl.ds(v0, n)],
                    slab_v.at[0, :, pl.ds(0, n)], sem0).wait()

                def red(j, _, dg=dg):
                    sl = pl.ds(j * L, L)
                    a = slab_v[0, 0, sl]
                    for d in range(1, 8):
                        a = a + slab_v[0, d, sl]
                    if dg == 0:
                        acc_v[sl] = a
                    else:
                        acc_v[sl] = acc_v[sl] + a
                    return _

                lax.fori_loop(0, n // L, red, None)
            pltpu.sync_copy(acc_v.at[pl.ds(0, n)],
                            s_h.at[pl.ds(f * VMAIN + v0, n)])

        for k in range(KMAX):
            uid = wid + NW * k
            f = uid // NVC
            vc = uid - f * NVC

            @pl.when(jnp.logical_and(uid < NU, vc < NVC - 1))
            def _():
                do_chunk(f, vc * CHUNK)

            @pl.when(jnp.logical_and(uid < NU, vc == NVC - 1))
            def _():
                do_tail(f, TAIL0, TAILN)

    return phase1


def _make_phase2(F, V, B, D):
    RPW = B // NW
    NVEC = RPW // L
    NST = F * RPW // 128
    NTAIL = V - VMAIN
    mesh = plsc.VectorSubcoreMesh(core_axis_name="c", subcore_axis_name="s")

    @functools.partial(
        pl.kernel,
        mesh=mesh,
        compiler_params=pltpu.CompilerParams(
            use_tc_tiling_on_sc=False, needs_layout_passes=False),
        out_type=jax.ShapeDtypeStruct((B,), jnp.float32),
        scratch_types=[
            pltpu.VMEM((F, RPW), jnp.int32),
            pltpu.VMEM((F * RPW,), jnp.int32),
            pltpu.VMEM((F * RPW,), jnp.float32),
            pltpu.VMEM((D, RPW), jnp.float32),
            pltpu.VMEM(((D + 1) * L,), jnp.float32),
            pltpu.VMEM((F * NTAIL,), jnp.float32),
            pltpu.VMEM((RPW,), jnp.float32),
            pltpu.SemaphoreType.DMA,
        ],
    )
    def phase2(s_h, stail_h, idx_h, den_h, w_h, out_h,
               idx_v, pos_v, buf_v, den_v, w_v, stail_v, out_v, sem):
        wid = lax.axis_index("s") * NC + lax.axis_index("c")
        b0 = wid * RPW
        pltpu.sync_copy(w_h, w_v)
        pltpu.sync_copy(stail_h, stail_v)
        pltpu.sync_copy(idx_h.at[:, pl.ds(b0, RPW)], idx_v)
        pltpu.sync_copy(den_h.at[:, pl.ds(b0, RPW)], den_v)

        vmax = jnp.full((L,), VMAIN - 1, jnp.int32)
        for f in range(F):
            off = jnp.full((L,), f * VMAIN, jnp.int32)

            def mkpos(j, _, off=off, f=f):
                sl = pl.ds(j * L, L)
                pos_v[pl.ds(f * RPW + j * L, L)] = (
                    jnp.minimum(idx_v[f, sl], vmax) + off)
                return _

            lax.fori_loop(0, NVEC, mkpos, None)

        def fire(bi, _):
            for j in range(13):
                st = (bi * 13 + j) * 128
                pltpu.make_async_copy(
                    s_h.at[pos_v.at[pl.ds(st, 128)]],
                    buf_v.at[pl.ds(st, 128)], sem).start()
            for j in range(13):
                st = (bi * 13 + j) * 128
                pltpu.make_async_copy(
                    s_h.at[pos_v.at[pl.ds(st, 128)]],
                    buf_v.at[pl.ds(st, 128)], sem).wait()
            return _

        lax.fori_loop(0, NST // 13, fire, None)

        wvecs = [w_v[pl.ds(dd * L, L)] for dd in range(D + 1)]
        vlim = jnp.full((L,), VMAIN, jnp.int32)
        zero = jnp.zeros((L,), jnp.int32)

        def red(g, _):
            sl = pl.ds(g * L, L)
            acc = wvecs[D]
            for dd in range(D):
                acc = acc + den_v[dd, sl] * wvecs[dd]
            for f in range(F):
                v = idx_v[f, sl]
                tv = plsc.load_gather(
                    stail_v, [jnp.maximum(v - vlim, zero) + (f * NTAIL)])
                acc = acc + jnp.where(
                    v >= vlim, tv, buf_v[pl.ds(f * RPW + g * L, L)])
            out_v[sl] = acc
            return _

        lax.fori_loop(0, NVEC, red, None)
        pltpu.sync_copy(out_v, out_h.at[pl.ds(b0, RPW)])

    return phase2


def kernel(sparse_idx, dense, tables, fc_w, fc_b):
    B, F = sparse_idx.shape
    Ft, V, E = tables.shape
    D = dense.shape[1]

    tab_t = jnp.swapaxes(tables, 1, 2)
    idx_flat = sparse_idx.astype(jnp.int32).T
    den_flat = dense.T
    w_splat = jnp.concatenate(
        [jnp.repeat(fc_w[:, 0], L), jnp.repeat(fc_b, L)])
    s_tail = jnp.sum(tables[:, VMAIN:, :], axis=-1).reshape(-1)

    s_flat = _make_phase1(F, V, E)(tab_t)
    out = _make_phase2(F, V, B, D)(
        s_flat, s_tail, idx_flat, den_flat, w_splat)
    return out[:, None]


# speedup vs baseline: 4.9030x; 1.0265x over previous
import functools

import jax
import jax.numpy as jnp
from jax import lax
from jax.experimental import pallas as pl
from jax.experimental.pallas import tpu as pltpu
from jax.experimental.pallas import tpu_sc as plsc

NC = 2
NS = 16
L = 16
NW = NC * NS

CHUNK = 8192
SUB = 2048
NSUB = CHUNK // SUB
VMAIN = 99968
NVC = 13
TAIL0 = 12 * CHUNK
TAILN = VMAIN - TAIL0


def _make_phase1(F, V, E):
    NU = F * NVC
    KMAX = (NU + NW - 1) // NW
    mesh = plsc.VectorSubcoreMesh(core_axis_name="c", subcore_axis_name="s")

    @functools.partial(
        pl.kernel,
        mesh=mesh,
        compiler_params=pltpu.CompilerParams(
            use_tc_tiling_on_sc=True, needs_layout_passes=False),
        out_type=jax.ShapeDtypeStruct((F * VMAIN,), jnp.float32),
        scratch_types=[
            pltpu.VMEM((2, 8, SUB), jnp.float32),
            pltpu.VMEM((CHUNK,), jnp.float32),
            pltpu.SemaphoreType.DMA,
            pltpu.SemaphoreType.DMA,
        ],
    )
    def phase1(tab_h, s_h, slab_v, acc_v, sem0, sem1):
        wid = lax.axis_index("s") * NC + lax.axis_index("c")
        sems = (sem0, sem1)

        def do_chunk(f, v0):
            def cp(i):
                dg, sub = divmod(i, NSUB)
                return pltpu.make_async_copy(
                    tab_h.at[f, pl.ds(dg * 8, 8),
                             pl.ds(v0 + sub * SUB, SUB)],
                    slab_v.at[i % 2], sems[i % 2])

            cp(0).start()
            for i in range(2 * NSUB):
                if i + 1 < 2 * NSUB:
                    cp(i + 1).start()
                cp(i).wait()
                dg, sub = divmod(i, NSUB)
                buf = i % 2

                def red(j, _, dg=dg, sub=sub, buf=buf):
                    for u in range(4):
                        sl = pl.ds(j * 4 * L + u * L, L)
                        a = slab_v[buf, 0, sl]
                        for d in range(1, 8):
                            a = a + slab_v[buf, d, sl]
                        asl = pl.ds(sub * SUB + j * 4 * L + u * L, L)
                        if dg == 0:
                            acc_v[asl] = a
                        else:
                            acc_v[asl] = acc_v[asl] + a
                    return _

                lax.fori_loop(0, SUB // (4 * L), red, None)
            pltpu.sync_copy(acc_v, s_h.at[pl.ds(f * VMAIN + v0, CHUNK)])

        def do_tail(f, v0, n):
            for dg in range(2):
                pltpu.make_async_copy(
                    tab_h.at[f, pl.ds(dg * 8, 8), pl.ds(v0, n)],
                    slab_v.at[0, :, pl.ds(0, n)], sem0).start()
                pltpu.make_async_copy(
                    tab_h.at[f, pl.ds(dg * 8, 8), pl.ds(v0, n)],
                    slab_v.at[0, :, pl.ds(0, n)], sem0).wait()

                def red(j, _, dg=dg):
                    sl = pl.ds(j * L, L)
                    a = slab_v[0, 0, sl]
                    for d in range(1, 8):
                        a = a + slab_v[0, d, sl]
                    if dg == 0:
                        acc_v[sl] = a
                    else:
                        acc_v[sl] = acc_v[sl] + a
                    return _

                lax.fori_loop(0, n // L, red, None)
            pltpu.sync_copy(acc_v.at[pl.ds(0, n)],
                            s_h.at[pl.ds(f * VMAIN + v0, n)])

        def kbody(k, _):
            uid = wid + NW * k
            f = uid // NVC
            vc = uid - f * NVC

            @pl.when(jnp.logical_and(uid < NU, vc < NVC - 1))
            def _():
                do_chunk(f, vc * CHUNK)

            @pl.when(jnp.logical_and(uid < NU, vc == NVC - 1))
            def _():
                do_tail(f, TAIL0, TAILN)

            return _

        lax.fori_loop(0, KMAX, kbody, None)

    return phase1


def _make_phase2(F, V, B, D):
    RPW = B // NW
    NVEC = RPW // L
    NST = F * RPW // 128
    NTAIL = V - VMAIN
    mesh = plsc.VectorSubcoreMesh(core_axis_name="c", subcore_axis_name="s")

    @functools.partial(
        pl.kernel,
        mesh=mesh,
        compiler_params=pltpu.CompilerParams(
            use_tc_tiling_on_sc=False, needs_layout_passes=False),
        out_type=jax.ShapeDtypeStruct((B,), jnp.float32),
        scratch_types=[
            pltpu.VMEM((F, RPW), jnp.int32),
            pltpu.VMEM((F * RPW,), jnp.int32),
            pltpu.VMEM((F * RPW,), jnp.float32),
            pltpu.VMEM((D, RPW), jnp.float32),
            pltpu.VMEM(((D + 1) * L,), jnp.float32),
            pltpu.VMEM((F * NTAIL,), jnp.float32),
            pltpu.VMEM((RPW,), jnp.float32),
            pltpu.SemaphoreType.DMA,
        ],
    )
    def phase2(s_h, stail_h, idx_h, den_h, w_h, out_h,
               idx_v, pos_v, buf_v, den_v, w_v, stail_v, out_v, sem):
        wid = lax.axis_index("s") * NC + lax.axis_index("c")
        b0 = wid * RPW
        pltpu.sync_copy(w_h, w_v)
        pltpu.sync_copy(stail_h, stail_v)
        pltpu.sync_copy(idx_h.at[:, pl.ds(b0, RPW)], idx_v)
        pltpu.sync_copy(den_h.at[:, pl.ds(b0, RPW)], den_v)

        vmax = jnp.full((L,), VMAIN - 1, jnp.int32)
        for f in range(F):
            off = jnp.full((L,), f * VMAIN, jnp.int32)

            def mkpos(j, _, off=off, f=f):
                sl = pl.ds(j * L, L)
                pos_v[pl.ds(f * RPW + j * L, L)] = (
                    jnp.minimum(idx_v[f, sl], vmax) + off)
                return _

            lax.fori_loop(0, NVEC, mkpos, None)

        def fire(bi, _):
            for j in range(13):
                st = (bi * 13 + j) * 128
                pltpu.make_async_copy(
                    s_h.at[pos_v.at[pl.ds(st, 128)]],
                    buf_v.at[pl.ds(st, 128)], sem).start()
            for j in range(13):
                st = (bi * 13 + j) * 128
                pltpu.make_async_copy(
                    s_h.at[pos_v.at[pl.ds(st, 128)]],
                    buf_v.at[pl.ds(st, 128)], sem).wait()
            return _

        lax.fori_loop(0, NST // 13, fire, None)

        wvecs = [w_v[pl.ds(dd * L, L)] for dd in range(D + 1)]
        vlim = jnp.full((L,), VMAIN, jnp.int32)
        zero = jnp.zeros((L,), jnp.int32)

        def red(g, _):
            sl = pl.ds(g * L, L)
            acc = wvecs[D]
            for dd in range(D):
                acc = acc + den_v[dd, sl] * wvecs[dd]
            for f in range(F):
                v = idx_v[f, sl]
                tv = plsc.load_gather(
                    stail_v, [jnp.maximum(v - vlim, zero) + (f * NTAIL)])
                acc = acc + jnp.where(
                    v >= vlim, tv, buf_v[pl.ds(f * RPW + g * L, L)])
            out_v[sl] = acc
            return _

        lax.fori_loop(0, NVEC, red, None)
        pltpu.sync_copy(out_v, out_h.at[pl.ds(b0, RPW)])

    return phase2


def kernel(sparse_idx, dense, tables, fc_w, fc_b):
    B, F = sparse_idx.shape
    Ft, V, E = tables.shape
    D = dense.shape[1]

    tab_t = jnp.swapaxes(tables, 1, 2)
    idx_flat = sparse_idx.astype(jnp.int32).T
    den_flat = dense.T
    w_splat = jnp.concatenate(
        [jnp.repeat(fc_w[:, 0], L), jnp.repeat(fc_b, L)])
    s_tail = jnp.sum(tables[:, VMAIN:, :], axis=-1).reshape(-1)

    s_flat = _make_phase1(F, V, E)(tab_t)
    out = _make_phase2(F, V, B, D)(
        s_flat, s_tail, idx_flat, den_flat, w_splat)
    return out[:, None]
